# baseline (speedup 1.0000x reference)
import functools

import jax
import jax.numpy as jnp
from jax.experimental import pallas as pl
from jax.experimental.pallas import tpu as pltpu

N, E = 1024, 2048
NFEAT_V, NFEAT_E, NHID, NCLASS = 128, 16, 64, 16
BN = 256
BE = 256


def _node_kernel(T_blk, T_full, adj_blk, Hv, He, W, p, b,
                 out_ref, d_scr, HW_scr, *, log_sm):
    i = pl.program_id(0)

    @pl.when(i == 0)
    def _init():
        d_scr[...] = jnp.sum(He[...] * p[...], axis=1).reshape(1, E)
        HW_scr[...] = jnp.dot(Hv[...], W[...],
                              preferred_element_type=jnp.float32)

    mult = jax.lax.dot_general(
        T_blk[...] * d_scr[...], T_full[...],
        (((1,), (1,)), ((), ())), preferred_element_type=jnp.float32)
    rows = i * BN + jax.lax.broadcasted_iota(jnp.int32, (BN, N), 0)
    cols = jax.lax.broadcasted_iota(jnp.int32, (BN, N), 1)
    A = jnp.where(rows == cols, 1.0, mult) * adj_blk[...]
    out = jnp.dot(A, HW_scr[...], preferred_element_type=jnp.float32) + b[...]
    if log_sm:
        m = jnp.max(out, axis=1, keepdims=True)
        out = out - (m + jnp.log(jnp.sum(jnp.exp(out - m), axis=1,
                                         keepdims=True)))
    else:
        out = jnp.maximum(out, 0.0)
    out_ref[...] = out


def _edge_kernel(T_cols, T_full, adj_blk, Hv, He, W, p, b,
                 out_ref, d_scr, HW_scr):
    j = pl.program_id(0)

    @pl.when(j == 0)
    def _init():
        d_scr[...] = jnp.sum(Hv[...] * p[...], axis=1, keepdims=True)
        HW_scr[...] = jnp.dot(jnp.maximum(He[...], 0.0), W[...],
                              preferred_element_type=jnp.float32)

    mult = jax.lax.dot_general(
        T_cols[...] * d_scr[...], T_full[...],
        (((0,), (0,)), ((), ())), preferred_element_type=jnp.float32)
    rows = j * BE + jax.lax.broadcasted_iota(jnp.int32, (BE, E), 0)
    cols = jax.lax.broadcasted_iota(jnp.int32, (BE, E), 1)
    A = jnp.where(rows == cols, 1.0, mult) * adj_blk[...]
    out = jnp.dot(A, HW_scr[...], preferred_element_type=jnp.float32) + b[...]
    out_ref[...] = jnp.maximum(out, 0.0)


def _node_call(T, adj_v, Hv, He, W, p, b, nin, nout, log_sm):
    return pl.pallas_call(
        functools.partial(_node_kernel, log_sm=log_sm),
        grid=(N // BN,),
        in_specs=[
            pl.BlockSpec((BN, E), lambda i: (i, 0)),
            pl.BlockSpec((N, E), lambda i: (0, 0)),
            pl.BlockSpec((BN, N), lambda i: (i, 0)),
            pl.BlockSpec((N, nin), lambda i: (0, 0)),
            pl.BlockSpec((E, NFEAT_E), lambda i: (0, 0)),
            pl.BlockSpec((nin, nout), lambda i: (0, 0)),
            pl.BlockSpec((1, NFEAT_E), lambda i: (0, 0)),
            pl.BlockSpec((1, nout), lambda i: (0, 0)),
        ],
        out_specs=pl.BlockSpec((BN, nout), lambda i: (i, 0)),
        out_shape=jax.ShapeDtypeStruct((N, nout), jnp.float32),
        scratch_shapes=[pltpu.VMEM((1, E), jnp.float32),
                        pltpu.VMEM((N, nout), jnp.float32)],
    )(T, T, adj_v, Hv, He, W, p, b)


def _edge_call(T, adj_e, Hv, He, W, p, b):
    return pl.pallas_call(
        _edge_kernel,
        grid=(E // BE,),
        in_specs=[
            pl.BlockSpec((N, BE), lambda j: (0, j)),
            pl.BlockSpec((N, E), lambda j: (0, 0)),
            pl.BlockSpec((BE, E), lambda j: (j, 0)),
            pl.BlockSpec((N, NHID), lambda j: (0, 0)),
            pl.BlockSpec((E, NFEAT_E), lambda j: (0, 0)),
            pl.BlockSpec((NFEAT_E, NFEAT_E), lambda j: (0, 0)),
            pl.BlockSpec((1, NHID), lambda j: (0, 0)),
            pl.BlockSpec((1, NFEAT_E), lambda j: (0, 0)),
        ],
        out_specs=pl.BlockSpec((BE, NFEAT_E), lambda j: (j, 0)),
        out_shape=jax.ShapeDtypeStruct((E, NFEAT_E), jnp.float32),
        scratch_shapes=[pltpu.VMEM((N, 1), jnp.float32),
                        pltpu.VMEM((E, NFEAT_E), jnp.float32)],
    )(T, T, adj_e, Hv, He, W, p, b)


def kernel(X, Z, adj_e, adj_v, T, W1, p1, b1, W2, p2, b2, W3, p3, b3):
    b1r, b2r, b3r = b1.reshape(1, -1), b2.reshape(1, -1), b3.reshape(1, -1)
    Xh = _node_call(T, adj_v, X, Z, W1, p1, b1r, NFEAT_V, NHID, log_sm=False)
    Zh = _edge_call(T, adj_e, Xh, Z, W2, p2, b2r)
    return _node_call(T, adj_v, Xh, Zh, W3, p3, b3r, NHID, NCLASS, log_sm=True)
